# baseline (speedup 1.0000x reference)
import functools

import jax
import jax.numpy as jnp
from jax import lax
from jax.experimental import pallas as pl
from jax.experimental.pallas import tpu as pltpu
from jax.experimental.pallas import tpu_sc as plsc

NUM_ENT = 100000
HDIM = 128
NUM_REL = 16
B = 16384

NC, NS, L = 2, 16, 16
NW = NC * NS
CHUNK = B // NW
SUB = 128
NSUB = CHUNK // SUB


def _body(x_hbm, y_hbm, r_hbm, tab_hbm, R_hbm, out_hbm,
          xi, yi, rv, Rv, xr, yr, sc, sem_x, sem_y):
    wid = lax.axis_index("s") * NC + lax.axis_index("c")
    base = wid * CHUNK

    pltpu.sync_copy(r_hbm.at[pl.ds(base, CHUNK)], rv)
    pltpu.sync_copy(R_hbm, Rv)

    rows0 = lax.broadcasted_iota(jnp.int32, (L,), 0)

    for sub in range(NSUB):
        off = base + sub * SUB
        pltpu.sync_copy(x_hbm.at[pl.ds(off, SUB)], xi)
        pltpu.sync_copy(y_hbm.at[pl.ds(off, SUB)], yi)
        cx = pltpu.async_copy(tab_hbm.at[xi], xr, sem_x)
        cy = pltpu.async_copy(tab_hbm.at[yi], yr, sem_y)
        cx.wait()
        cy.wait()
        for g in range(SUB // L):
            goff = sub * SUB + g * L
            rvec = rv[pl.ds(goff, L)]
            rows = rows0 + g * L

            def dstep(d, acc):
                col = jnp.full((L,), d, jnp.int32)
                gx = plsc.load_gather(xr, [rows, col])
                gy = plsc.load_gather(yr, [rows, col])
                gr = plsc.load_gather(Rv, [rvec, col])
                return acc + gx * gy * gr

            acc = lax.fori_loop(0, HDIM, dstep, jnp.zeros((L,), jnp.float32),
                                unroll=4)
            sc[pl.ds(goff, L)] = acc

    pltpu.sync_copy(sc, out_hbm.at[pl.ds(base, CHUNK)])


@jax.jit
def kernel(x, y, r, emb_table, R):
    mesh = plsc.VectorSubcoreMesh(core_axis_name="c", subcore_axis_name="s")
    return pl.kernel(
        _body,
        out_type=jax.ShapeDtypeStruct((B,), jnp.float32),
        mesh=mesh,
        compiler_params=pltpu.CompilerParams(needs_layout_passes=False),
        scratch_types=[
            pltpu.VMEM((SUB,), jnp.int32),
            pltpu.VMEM((SUB,), jnp.int32),
            pltpu.VMEM((CHUNK,), jnp.int32),
            pltpu.VMEM((NUM_REL, HDIM), jnp.float32),
            pltpu.VMEM((SUB, HDIM), jnp.float32),
            pltpu.VMEM((SUB, HDIM), jnp.float32),
            pltpu.VMEM((CHUNK,), jnp.float32),
            pltpu.SemaphoreType.DMA,
            pltpu.SemaphoreType.DMA,
        ],
    )(x, y, r, emb_table, R)


# speedup vs baseline: 1.7534x; 1.7534x over previous
import jax
import jax.numpy as jnp
from jax import lax
from jax.experimental import pallas as pl
from jax.experimental.pallas import tpu as pltpu
from jax.experimental.pallas import tpu_sc as plsc

NUM_ENT = 100000
HDIM = 128
NUM_REL = 16
B = 16384

NC, NS, L = 2, 16, 16
NW = NC * NS
CHUNK = B // NW
SUB = 128
NSUB = CHUNK // SUB
NBLK = HDIM // L


def _body(x_hbm, y_hbm, r_hbm, tab_hbm, R_hbm, out_hbm,
          xi0, xi1, yi0, yi1, rv, Rv, xr0, xr1, yr0, yr1, sc,
          sx0, sx1, sy0, sy1):
    wid = lax.axis_index("s") * NC + lax.axis_index("c")
    base = wid * CHUNK
    xis, yis = [xi0, xi1], [yi0, yi1]
    xrs, yrs = [xr0, xr1], [yr0, yr1]
    sxs, sys_ = [sx0, sx1], [sy0, sy1]

    pltpu.sync_copy(r_hbm.at[pl.ds(base, CHUNK)], rv)
    pltpu.sync_copy(R_hbm, Rv)

    def start(sub):
        k = sub % 2
        off = base + sub * SUB
        pltpu.sync_copy(x_hbm.at[pl.ds(off, SUB)], xis[k])
        pltpu.sync_copy(y_hbm.at[pl.ds(off, SUB)], yis[k])
        cx = pltpu.async_copy(tab_hbm.at[xis[k]], xrs[k], sxs[k])
        cy = pltpu.async_copy(tab_hbm.at[yis[k]], yrs[k], sys_[k])
        return cx, cy

    lane = lax.broadcasted_iota(jnp.int32, (L,), 0)
    pend = start(0)
    for sub in range(NSUB):
        k = sub % 2
        cx, cy = pend
        if sub + 1 < NSUB:
            pend = start(sub + 1)
        cx.wait()
        cy.wait()
        xr, yr = xrs[k], yrs[k]

        def gbody(g, _, xr=xr, yr=yr, sub=sub):
            goff = g * L
            rvec = rv[pl.ds(sub * SUB + goff, L)]
            out = jnp.zeros((L,), jnp.float32)
            for j in range(L):
                e = goff + j
                re = rvec[j]
                acc = (xr[e, pl.ds(0, L)] * yr[e, pl.ds(0, L)]
                       * Rv[re, pl.ds(0, L)])
                for blk in range(1, NBLK):
                    acc = acc + (xr[e, pl.ds(blk * L, L)]
                                 * yr[e, pl.ds(blk * L, L)]
                                 * Rv[re, pl.ds(blk * L, L)])
                s = jnp.sum(acc)
                out = jnp.where(lane == j, s, out)
            sc[pl.ds(sub * SUB + goff, L)] = out
            return 0

        lax.fori_loop(0, SUB // L, gbody, 0)

    pltpu.sync_copy(sc, out_hbm.at[pl.ds(base, CHUNK)])


@jax.jit
def kernel(x, y, r, emb_table, R):
    mesh = plsc.VectorSubcoreMesh(core_axis_name="c", subcore_axis_name="s")
    return pl.kernel(
        _body,
        out_type=jax.ShapeDtypeStruct((B,), jnp.float32),
        mesh=mesh,
        compiler_params=pltpu.CompilerParams(needs_layout_passes=False),
        scratch_types=[
            pltpu.VMEM((SUB,), jnp.int32),
            pltpu.VMEM((SUB,), jnp.int32),
            pltpu.VMEM((SUB,), jnp.int32),
            pltpu.VMEM((SUB,), jnp.int32),
            pltpu.VMEM((CHUNK,), jnp.int32),
            pltpu.VMEM((NUM_REL, HDIM), jnp.float32),
            pltpu.VMEM((SUB, HDIM), jnp.float32),
            pltpu.VMEM((SUB, HDIM), jnp.float32),
            pltpu.VMEM((SUB, HDIM), jnp.float32),
            pltpu.VMEM((SUB, HDIM), jnp.float32),
            pltpu.VMEM((CHUNK,), jnp.float32),
            pltpu.SemaphoreType.DMA,
            pltpu.SemaphoreType.DMA,
            pltpu.SemaphoreType.DMA,
            pltpu.SemaphoreType.DMA,
        ],
    )(x, y, r, emb_table, R)
